# baseline (speedup 1.0000x reference)
import functools

import jax
import jax.numpy as jnp
from jax import lax
from jax.experimental import pallas as pl
from jax.experimental.pallas import tpu as pltpu
from jax.experimental.pallas import tpu_sc as plsc

T = 2048
H = 768
FF = 1024
E = 64
CAP = 192
S = E * CAP
SPAD = S + 8
NC, NS = 2, 16
NW = NC * NS
BPW = T // NW
H2 = H // 2
RCHUNK = 512


def _router_body(x_ref, wg_ref, slot_s_ref, slot_g_ref, gate_ref, xb16_ref):
    x = x_ref[...]
    logits = jnp.dot(x, wg_ref[...], preferred_element_type=jnp.float32)
    m = jnp.max(logits, axis=1, keepdims=True)
    ex = jnp.exp(logits - m)
    gate = 1.0 / jnp.sum(ex, axis=1, keepdims=True)
    col = lax.broadcasted_iota(jnp.int32, (T, E), 1)
    idx = jnp.min(jnp.where(logits == m, col, E), axis=1, keepdims=True)
    oh = (col == idx).astype(jnp.float32)
    cum = oh
    k = 1
    while k < T:
        cum = cum + jnp.concatenate(
            [jnp.zeros((k, E), jnp.float32), cum[:T - k]], axis=0)
        k *= 2
    cum = cum - oh
    pos = jnp.sum(cum * oh, axis=1, keepdims=True).astype(jnp.int32)
    keep = pos < CAP
    slot = idx * CAP + pos
    slot_s_ref[...] = jnp.where(keep, slot, S)
    slot_g_ref[...] = jnp.where(keep, slot, S - 1)
    gate_ref[...] = jnp.where(keep, gate, 0.0)
    xb16_ref[...] = x.astype(jnp.bfloat16)


_router = pl.pallas_call(
    _router_body,
    out_shape=(
        jax.ShapeDtypeStruct((T, 1), jnp.int32),
        jax.ShapeDtypeStruct((T, 1), jnp.int32),
        jax.ShapeDtypeStruct((T, 1), jnp.float32),
        jax.ShapeDtypeStruct((T, H), jnp.bfloat16),
    ),
)


@functools.cache
def _sc_kernels():
    mesh = plsc.VectorSubcoreMesh(
        core_axis_name="c", subcore_axis_name="s", num_cores=NC, num_subcores=NS)
    scratch = [
        pltpu.VMEM((BPW,), jnp.int32),
        pltpu.VMEM((BPW, H2), jnp.int32),
        pltpu.SemaphoreType.DMA,
    ]

    @functools.partial(
        pl.kernel,
        out_type=jax.ShapeDtypeStruct((SPAD, H2), jnp.int32),
        mesh=mesh,
        scratch_types=scratch,
    )
    def dispatch(x_hbm, slot_hbm, xe_hbm, idx_v, rows_v, sem):
        wid = lax.axis_index("s") * NC + lax.axis_index("c")
        base = wid * BPW
        pltpu.sync_copy(slot_hbm.at[pl.ds(base, BPW)], idx_v)
        pltpu.sync_copy(x_hbm.at[pl.ds(base, BPW)], rows_v)
        pltpu.async_copy(rows_v, xe_hbm.at[idx_v], sem).wait()

    @functools.partial(
        pl.kernel,
        out_type=jax.ShapeDtypeStruct((T, H2), jnp.int32),
        mesh=mesh,
        scratch_types=scratch,
    )
    def combine(ye_hbm, slot_hbm, y_hbm, idx_v, rows_v, sem):
        wid = lax.axis_index("s") * NC + lax.axis_index("c")
        base = wid * BPW
        pltpu.sync_copy(slot_hbm.at[pl.ds(base, BPW)], idx_v)
        pltpu.async_copy(ye_hbm.at[idx_v], rows_v, sem).wait()
        pltpu.sync_copy(rows_v, y_hbm.at[pl.ds(base, BPW)])

    return dispatch, combine


CT = 64
NT = CAP // CT


def _ffn_body(xe_ref, w1_ref, b1_ref, w2_ref, b2_ref, ye_ref):
    xb = xe_ref[...]
    a = jnp.dot(xb, w1_ref[0].astype(jnp.bfloat16),
                preferred_element_type=jnp.float32) + b1_ref[0]
    h = a * (1.0 / (1.0 + jnp.exp(-a)))
    y = jnp.dot(h.astype(jnp.bfloat16), w2_ref[0].astype(jnp.bfloat16),
                preferred_element_type=jnp.float32) + b2_ref[0]
    ye_ref[...] = y.astype(jnp.bfloat16)


_ffn = pl.pallas_call(
    _ffn_body,
    grid=(E,),
    in_specs=[
        pl.BlockSpec((CAP, H), lambda e: (e, 0)),
        pl.BlockSpec((1, H, FF), lambda e: (e, 0, 0)),
        pl.BlockSpec((1, 1, FF), lambda e: (e, 0, 0)),
        pl.BlockSpec((1, FF, H), lambda e: (e, 0, 0)),
        pl.BlockSpec((1, 1, H), lambda e: (e, 0, 0)),
    ],
    out_specs=pl.BlockSpec((CAP, H), lambda e: (e, 0)),
    out_shape=jax.ShapeDtypeStruct((S, H), jnp.bfloat16),
)


def _scale_body(yr_ref, g_ref, out_ref):
    g = g_ref[...]
    out_ref[...] = jnp.where(g > 0.0, yr_ref[...].astype(jnp.float32) * g, 0.0)


_scale = pl.pallas_call(
    _scale_body,
    out_shape=jax.ShapeDtypeStruct((T, H), jnp.float32),
)


def kernel(hidden_states, Wg, W1, b1, W2, b2):
    orig_shape = hidden_states.shape
    x = hidden_states.reshape(T, H)
    dispatch, combine = _sc_kernels()
    slot_s, slot_g, gate, xb16 = _router(x, Wg)
    x_pairs = lax.bitcast_convert_type(xb16.reshape(T, H2, 2), jnp.int32)
    xe_i32 = dispatch(x_pairs, slot_s.reshape(T))
    xe16 = lax.bitcast_convert_type(xe_i32, jnp.bfloat16).reshape(SPAD, H)
    ye = _ffn(xe16, W1, b1.reshape(E, 1, FF), W2, b2.reshape(E, 1, H))
    ye_i32 = lax.bitcast_convert_type(ye.reshape(S, H2, 2), jnp.int32)
    yr_i32 = combine(ye_i32, slot_g.reshape(T))
    y_raw = lax.bitcast_convert_type(yr_i32, jnp.bfloat16).reshape(T, H)
    y = _scale(y_raw, gate)
    return y.reshape(orig_shape)


# speedup vs baseline: 3.8221x; 3.8221x over previous
import functools

import jax
import jax.numpy as jnp
from jax import lax
from jax.experimental import pallas as pl
from jax.experimental.pallas import tpu as pltpu
from jax.experimental.pallas import tpu_sc as plsc

T = 2048
H = 768
FF = 1024
E = 64
CAP = 192
S = E * CAP
SPAD = S + 8
NC, NS = 2, 16
NW = NC * NS
BPW = T // NW
H2 = H // 2
RCHUNK = 512


def _b16r(v):
    b = lax.bitcast_convert_type(v, jnp.int32)
    return b + 0x7FFF + (lax.shift_right_logical(b, 16) & 1)


def _pack_halves(v):
    lo = _b16r(v[:, :H2])
    hi = _b16r(v[:, H2:])
    return lax.shift_right_logical(lo, 16) | (hi & jnp.int32(-65536))


def _unpack_halves(w):
    lo = lax.bitcast_convert_type(lax.shift_left(w, 16), jnp.float32)
    hi = lax.bitcast_convert_type(w & jnp.int32(-65536), jnp.float32)
    return lo, hi


def _router_body(x_ref, wg_ref, slot_s_ref, slot_g_ref, gate_ref, xpk_ref):
    x = x_ref[...]
    logits = jnp.dot(x, wg_ref[...], preferred_element_type=jnp.float32)
    m = jnp.max(logits, axis=1, keepdims=True)
    ex = jnp.exp(logits - m)
    gate = 1.0 / jnp.sum(ex, axis=1, keepdims=True)
    col = lax.broadcasted_iota(jnp.int32, (T, E), 1)
    idx = jnp.min(jnp.where(logits == m, col, E), axis=1, keepdims=True)
    oh = (col == idx).astype(jnp.float32)
    cum = oh
    k = 1
    while k < T:
        cum = cum + jnp.concatenate(
            [jnp.zeros((k, E), jnp.float32), cum[:T - k]], axis=0)
        k *= 2
    cum = cum - oh
    pos = jnp.sum(cum * oh, axis=1, keepdims=True).astype(jnp.int32)
    keep = pos < CAP
    slot = idx * CAP + pos
    slot_s_ref[...] = jnp.where(keep, slot, S)
    slot_g_ref[...] = jnp.where(keep, slot, S - 1)
    gate_ref[...] = jnp.where(keep, gate, 0.0)
    xpk_ref[...] = _pack_halves(x)


_router = pl.pallas_call(
    _router_body,
    out_shape=(
        jax.ShapeDtypeStruct((T, 1), jnp.int32),
        jax.ShapeDtypeStruct((T, 1), jnp.int32),
        jax.ShapeDtypeStruct((T, 1), jnp.float32),
        jax.ShapeDtypeStruct((T, H2), jnp.int32),
    ),
)


@functools.cache
def _sc_kernels():
    mesh = plsc.VectorSubcoreMesh(
        core_axis_name="c", subcore_axis_name="s", num_cores=NC, num_subcores=NS)
    scratch = [
        pltpu.VMEM((BPW,), jnp.int32),
        pltpu.VMEM((BPW, H2), jnp.int32),
        pltpu.SemaphoreType.DMA,
    ]

    @functools.partial(
        pl.kernel,
        out_type=jax.ShapeDtypeStruct((SPAD, H2), jnp.int32),
        mesh=mesh,
        scratch_types=scratch,
    )
    def dispatch(x_hbm, slot_hbm, xe_hbm, idx_v, rows_v, sem):
        wid = lax.axis_index("s") * NC + lax.axis_index("c")
        base = wid * BPW
        pltpu.sync_copy(slot_hbm.at[pl.ds(base, BPW)], idx_v)
        pltpu.sync_copy(x_hbm.at[pl.ds(base, BPW)], rows_v)
        pltpu.async_copy(rows_v, xe_hbm.at[idx_v], sem).wait()

    @functools.partial(
        pl.kernel,
        out_type=jax.ShapeDtypeStruct((T, H2), jnp.int32),
        mesh=mesh,
        scratch_types=scratch,
    )
    def combine(ye_hbm, slot_hbm, y_hbm, idx_v, rows_v, sem):
        wid = lax.axis_index("s") * NC + lax.axis_index("c")
        base = wid * BPW
        pltpu.sync_copy(slot_hbm.at[pl.ds(base, BPW)], idx_v)
        pltpu.async_copy(ye_hbm.at[idx_v], rows_v, sem).wait()
        pltpu.sync_copy(rows_v, y_hbm.at[pl.ds(base, BPW)])

    return dispatch, combine


CT = 64
NT = CAP // CT


def _ffn_body(xe_ref, w1_ref, b1_ref, w2_ref, b2_ref, ye_ref):
    lo, hi = _unpack_halves(xe_ref[...])
    a = (jnp.dot(lo.astype(jnp.bfloat16), w1_ref[0, :H2].astype(jnp.bfloat16),
                 preferred_element_type=jnp.float32)
         + jnp.dot(hi.astype(jnp.bfloat16), w1_ref[0, H2:].astype(jnp.bfloat16),
                   preferred_element_type=jnp.float32)
         + b1_ref[0])
    h = a * (1.0 / (1.0 + jnp.exp(-a)))
    y = jnp.dot(h.astype(jnp.bfloat16), w2_ref[0].astype(jnp.bfloat16),
                preferred_element_type=jnp.float32) + b2_ref[0]
    ye_ref[...] = _pack_halves(y)


_ffn = pl.pallas_call(
    _ffn_body,
    grid=(E,),
    in_specs=[
        pl.BlockSpec((CAP, H2), lambda e: (e, 0)),
        pl.BlockSpec((1, H, FF), lambda e: (e, 0, 0)),
        pl.BlockSpec((1, 1, FF), lambda e: (e, 0, 0)),
        pl.BlockSpec((1, FF, H), lambda e: (e, 0, 0)),
        pl.BlockSpec((1, 1, H), lambda e: (e, 0, 0)),
    ],
    out_specs=pl.BlockSpec((CAP, H2), lambda e: (e, 0)),
    out_shape=jax.ShapeDtypeStruct((S, H2), jnp.int32),
)


def _scale_body(yr_ref, g_ref, out_ref):
    g = g_ref[...]
    lo, hi = _unpack_halves(yr_ref[...])
    out_ref[:, :H2] = jnp.where(g > 0.0, lo * g, 0.0)
    out_ref[:, H2:] = jnp.where(g > 0.0, hi * g, 0.0)


_scale = pl.pallas_call(
    _scale_body,
    out_shape=jax.ShapeDtypeStruct((T, H), jnp.float32),
)


def kernel(hidden_states, Wg, W1, b1, W2, b2):
    orig_shape = hidden_states.shape
    x = hidden_states.reshape(T, H)
    dispatch, combine = _sc_kernels()
    slot_s, slot_g, gate, xpk = _router(x, Wg)
    xe = dispatch(xpk, slot_s.reshape(T))
    ye = _ffn(xe, W1, b1.reshape(E, 1, FF), W2, b2.reshape(E, 1, H))
    y_raw = combine(ye, slot_g.reshape(T))
    y = _scale(y_raw, gate)
    return y.reshape(orig_shape)


# speedup vs baseline: 3.8239x; 1.0005x over previous
import functools

import jax
import jax.numpy as jnp
from jax import lax
from jax.experimental import pallas as pl
from jax.experimental.pallas import tpu as pltpu
from jax.experimental.pallas import tpu_sc as plsc

T = 2048
H = 768
FF = 1024
E = 64
CAP = 192
S = E * CAP
SPAD = S + 8
NC, NS = 2, 16
NW = NC * NS
BPW = T // NW
H2 = H // 2


def _b16r(v):
    b = lax.bitcast_convert_type(v, jnp.int32)
    return b + 0x7FFF + (lax.shift_right_logical(b, 16) & 1)


def _pack_halves(v):
    lo = _b16r(v[:, :H2])
    hi = _b16r(v[:, H2:])
    return lax.shift_right_logical(lo, 16) | (hi & jnp.int32(-65536))


def _unpack_halves(w):
    lo = lax.bitcast_convert_type(lax.shift_left(w, 16), jnp.float32)
    hi = lax.bitcast_convert_type(w & jnp.int32(-65536), jnp.float32)
    return lo, hi


def _router_body(x_ref, wg_ref, slot_s_ref, slot_g_ref, gate_ref, xpk_ref):
    x = x_ref[...]
    logits = jnp.dot(x, wg_ref[...], preferred_element_type=jnp.float32)
    m = jnp.max(logits, axis=1, keepdims=True)
    ex = jnp.exp(logits - m)
    gate = 1.0 / jnp.sum(ex, axis=1, keepdims=True)
    col = lax.broadcasted_iota(jnp.int32, (T, E), 1)
    idx = jnp.min(jnp.where(logits == m, col, E), axis=1, keepdims=True)
    oh = (col == idx).astype(jnp.float32)
    cum = oh
    k = 1
    while k < T:
        cum = cum + jnp.concatenate(
            [jnp.zeros((k, E), jnp.float32), cum[:T - k]], axis=0)
        k *= 2
    cum = cum - oh
    pos = jnp.sum(cum * oh, axis=1, keepdims=True).astype(jnp.int32)
    keep = pos < CAP
    slot = idx * CAP + pos
    slot_s_ref[...] = jnp.where(keep, slot, S)
    slot_g_ref[...] = jnp.where(keep, slot, S - 1)
    gate_ref[...] = jnp.where(keep, gate, 0.0)
    xpk_ref[...] = _pack_halves(x)


_router = pl.pallas_call(
    _router_body,
    out_shape=(
        jax.ShapeDtypeStruct((T, 1), jnp.int32),
        jax.ShapeDtypeStruct((T, 1), jnp.int32),
        jax.ShapeDtypeStruct((T, 1), jnp.float32),
        jax.ShapeDtypeStruct((T, H2), jnp.int32),
    ),
)


@functools.cache
def _sc_kernels():
    mesh = plsc.VectorSubcoreMesh(
        core_axis_name="c", subcore_axis_name="s", num_cores=NC, num_subcores=NS)
    scratch = [
        pltpu.VMEM((BPW,), jnp.int32),
        pltpu.VMEM((BPW, H2), jnp.int32),
        pltpu.SemaphoreType.DMA,
    ]

    @functools.partial(
        pl.kernel,
        out_type=jax.ShapeDtypeStruct((SPAD, H2), jnp.int32),
        mesh=mesh,
        scratch_types=scratch,
    )
    def dispatch(x_hbm, slot_hbm, xe_hbm, idx_v, rows_v, sem):
        wid = lax.axis_index("s") * NC + lax.axis_index("c")
        base = wid * BPW
        pltpu.sync_copy(slot_hbm.at[pl.ds(base, BPW)], idx_v)
        pltpu.sync_copy(x_hbm.at[pl.ds(base, BPW)], rows_v)
        pltpu.async_copy(rows_v, xe_hbm.at[idx_v], sem).wait()

    @functools.partial(
        pl.kernel,
        out_type=jax.ShapeDtypeStruct((T, H2), jnp.int32),
        mesh=mesh,
        scratch_types=scratch,
    )
    def combine(ye_hbm, slot_hbm, y_hbm, idx_v, rows_v, sem):
        wid = lax.axis_index("s") * NC + lax.axis_index("c")
        base = wid * BPW
        pltpu.sync_copy(slot_hbm.at[pl.ds(base, BPW)], idx_v)
        pltpu.async_copy(ye_hbm.at[idx_v], rows_v, sem).wait()
        pltpu.sync_copy(rows_v, y_hbm.at[pl.ds(base, BPW)])

    return dispatch, combine


def _ffn_body(xe_ref, w1_ref, b1_ref, w2_ref, b2_ref, ye_ref):
    lo, hi = _unpack_halves(xe_ref[...])
    a = (jnp.dot(lo.astype(jnp.bfloat16), w1_ref[0, :H2].astype(jnp.bfloat16),
                 preferred_element_type=jnp.float32)
         + jnp.dot(hi.astype(jnp.bfloat16), w1_ref[0, H2:].astype(jnp.bfloat16),
                   preferred_element_type=jnp.float32)
         + b1_ref[0])
    h = a * (1.0 / (1.0 + jnp.exp(-a)))
    y = jnp.dot(h.astype(jnp.bfloat16), w2_ref[0].astype(jnp.bfloat16),
                preferred_element_type=jnp.float32) + b2_ref[0]
    ye_ref[...] = _pack_halves(y)


_ffn = pl.pallas_call(
    _ffn_body,
    grid=(E,),
    in_specs=[
        pl.BlockSpec((CAP, H2), lambda e: (e, 0)),
        pl.BlockSpec((1, H, FF), lambda e: (e, 0, 0)),
        pl.BlockSpec((1, 1, FF), lambda e: (e, 0, 0)),
        pl.BlockSpec((1, FF, H), lambda e: (e, 0, 0)),
        pl.BlockSpec((1, 1, H), lambda e: (e, 0, 0)),
    ],
    out_specs=pl.BlockSpec((CAP, H2), lambda e: (e, 0)),
    out_shape=jax.ShapeDtypeStruct((S, H2), jnp.int32),
)


def _scale_body(yr_ref, g_ref, out_ref):
    g = g_ref[...]
    lo, hi = _unpack_halves(yr_ref[...])
    out_ref[:, :H2] = jnp.where(g > 0.0, lo * g, 0.0)
    out_ref[:, H2:] = jnp.where(g > 0.0, hi * g, 0.0)


_scale = pl.pallas_call(
    _scale_body,
    out_shape=jax.ShapeDtypeStruct((T, H), jnp.float32),
)


def kernel(hidden_states, Wg, W1, b1, W2, b2):
    orig_shape = hidden_states.shape
    x = hidden_states.reshape(T, H)
    dispatch, combine = _sc_kernels()
    slot_s, slot_g, gate, xpk = _router(x, Wg)
    xe = dispatch(xpk, slot_s.reshape(T))
    ye = _ffn(xe, W1, b1.reshape(E, 1, FF), W2, b2.reshape(E, 1, H))
    y_raw = combine(ye, slot_g.reshape(T))
    y = _scale(y_raw, gate)
    return y.reshape(orig_shape)
